# baseline (speedup 1.0000x reference)
import functools

import jax
import jax.numpy as jnp
from jax import lax
from jax.experimental import pallas as pl
from jax.experimental.pallas import tpu as pltpu
from jax.experimental.pallas import tpu_sc as plsc

_VOCAB = 100000
_EMB = 64
_BATCH = 4096
_SEQ = 200
_NOUT = 10

_NC = 2
_NS = 16
_NW = _NC * _NS
_ROWS_PER_W = _BATCH // _NW
_CA = 120
_CB = 80
_IDX_PER_W = _ROWS_PER_W * _SEQ
_LOOK = 4


def _sc_pool(idx_rs, emb_table):
    mesh = plsc.VectorSubcoreMesh(core_axis_name="c", subcore_axis_name="s")

    @functools.partial(
        pl.kernel,
        mesh=mesh,
        out_type=jax.ShapeDtypeStruct((_BATCH * _EMB,), jnp.float32),
        scratch_types=[
            pltpu.VMEM((_ROWS_PER_W, _SEQ), jnp.int32),
            pltpu.VMEM((_ROWS_PER_W * _EMB,), jnp.float32),
        ] + [pltpu.VMEM((_CA, _EMB), jnp.float32)] * _LOOK
          + [pltpu.VMEM((_CB, _EMB), jnp.float32)] * _LOOK
          + [pltpu.SemaphoreType.DMA] * (2 * _LOOK),
        compiler_params=pltpu.CompilerParams(use_tc_tiling_on_sc=False),
    )
    def pool_kernel(idx_hbm, table_hbm, out_hbm, idx_v, outs_v, *bufsem):
        bufA = bufsem[:_LOOK]
        bufB = bufsem[_LOOK:2 * _LOOK]
        semA = bufsem[2 * _LOOK:3 * _LOOK]
        semB = bufsem[3 * _LOOK:]
        wid = lax.axis_index("s") * _NC + lax.axis_index("c")

        pltpu.sync_copy(idx_hbm.at[pl.ds(wid * _ROWS_PER_W, _ROWS_PER_W)],
                        idx_v)

        def cpA(row, p):
            return pltpu.make_async_copy(
                table_hbm.at[idx_v.at[row, pl.ds(0, _CA)]],
                bufA[p], semA[p])

        def cpB(row, p):
            return pltpu.make_async_copy(
                table_hbm.at[idx_v.at[row, pl.ds(_CA, _CB)]],
                bufB[p], semB[p])

        for p in range(_LOOK):
            cpA(p, p).start()
            cpB(p, p).start()

        neg = jnp.full((16,), -jnp.inf, dtype=jnp.float32)

        def reduce_chunk(buf, n, accs):
            def body(t, accs):
                a0, a1, a2, a3 = accs
                for u in range(8):
                    r = t * 8 + u
                    a0 = jnp.maximum(a0, buf[r, pl.ds(0, 16)])
                    a1 = jnp.maximum(a1, buf[r, pl.ds(16, 16)])
                    a2 = jnp.maximum(a2, buf[r, pl.ds(32, 16)])
                    a3 = jnp.maximum(a3, buf[r, pl.ds(48, 16)])
                return a0, a1, a2, a3
            return lax.fori_loop(0, n // 8, body, accs)

        def group(g, carry):
            for p in range(_LOOK):
                row = _LOOK * g + p
                cpA(row, p).wait()
                accs = reduce_chunk(bufA[p], _CA, (neg, neg, neg, neg))
                cpB(row, p).wait()
                accs = reduce_chunk(bufB[p], _CB, accs)

                @pl.when(row + _LOOK < _ROWS_PER_W)
                def _start_next():
                    cpA(row + _LOOK, p).start()
                    cpB(row + _LOOK, p).start()

                a0, a1, a2, a3 = accs
                outs_v[pl.ds(row * _EMB, 16)] = a0
                outs_v[pl.ds(row * _EMB + 16, 16)] = a1
                outs_v[pl.ds(row * _EMB + 32, 16)] = a2
                outs_v[pl.ds(row * _EMB + 48, 16)] = a3
            return carry

        lax.fori_loop(0, _ROWS_PER_W // _LOOK, group, 0)

        pltpu.sync_copy(outs_v,
                        out_hbm.at[pl.ds(wid * _ROWS_PER_W * _EMB,
                                         _ROWS_PER_W * _EMB)])

    return pool_kernel


def _tc_head(pooled, W_out, b_out):
    def body(x_ref, w_ref, b_ref, o_ref):
        logits = jnp.dot(x_ref[...], w_ref[...],
                         preferred_element_type=jnp.float32) + b_ref[...]
        m = jnp.max(logits, axis=-1, keepdims=True)
        e = jnp.exp(logits - m)
        o_ref[...] = e / jnp.sum(e, axis=-1, keepdims=True)

    return pl.pallas_call(
        body,
        out_shape=jax.ShapeDtypeStruct((_BATCH, _NOUT), jnp.float32),
    )(pooled, W_out, b_out.reshape(1, _NOUT))


def kernel(indices, emb_table, W_out, b_out):
    pooled_flat = _sc_pool(indices, emb_table)(indices, emb_table)
    pooled = pooled_flat.reshape(_BATCH, _EMB)
    return _tc_head(pooled, W_out, b_out)


# speedup vs baseline: 1.0117x; 1.0117x over previous
import functools

import jax
import jax.numpy as jnp
from jax import lax
from jax.experimental import pallas as pl
from jax.experimental.pallas import tpu as pltpu
from jax.experimental.pallas import tpu_sc as plsc

_VOCAB = 100000
_EMB = 64
_BATCH = 4096
_SEQ = 200
_NOUT = 10

_NC = 2
_NS = 16
_NW = _NC * _NS
_ROWS_PER_W = _BATCH // _NW
_CA = 128
_CB = 72
_IDX_PER_W = _ROWS_PER_W * _SEQ
_LOOK = 4


def _sc_pool(idx_rs, emb_table):
    mesh = plsc.VectorSubcoreMesh(core_axis_name="c", subcore_axis_name="s")

    @functools.partial(
        pl.kernel,
        mesh=mesh,
        out_type=jax.ShapeDtypeStruct((_BATCH * _EMB,), jnp.float32),
        scratch_types=[
            pltpu.VMEM((_ROWS_PER_W, _CA), jnp.int32),
            pltpu.VMEM((_ROWS_PER_W, _CA), jnp.int32),
            pltpu.VMEM((_ROWS_PER_W * _EMB,), jnp.float32),
        ] + [pltpu.VMEM((_CA, _EMB), jnp.float32)] * _LOOK
          + [pltpu.VMEM((_CB, _EMB), jnp.float32)] * _LOOK
          + [pltpu.SemaphoreType.DMA] * (2 * _LOOK),
        compiler_params=pltpu.CompilerParams(use_tc_tiling_on_sc=False),
    )
    def pool_kernel(idxA_hbm, idxB_hbm, table_hbm, out_hbm,
                    idxA_v, idxB_v, outs_v, *bufsem):
        bufA = bufsem[:_LOOK]
        bufB = bufsem[_LOOK:2 * _LOOK]
        semA = bufsem[2 * _LOOK:3 * _LOOK]
        semB = bufsem[3 * _LOOK:]
        wid = lax.axis_index("s") * _NC + lax.axis_index("c")

        pltpu.sync_copy(idxA_hbm.at[pl.ds(wid * _ROWS_PER_W, _ROWS_PER_W)],
                        idxA_v)
        pltpu.sync_copy(idxB_hbm.at[pl.ds(wid * _ROWS_PER_W, _ROWS_PER_W)],
                        idxB_v)

        def cpA(row, p):
            return pltpu.make_async_copy(
                table_hbm.at[idxA_v.at[row, pl.ds(0, _CA)]],
                bufA[p], semA[p])

        def cpB(row, p):
            return pltpu.make_async_copy(
                table_hbm.at[idxB_v.at[row, pl.ds(0, _CB)]],
                bufB[p], semB[p])

        for p in range(_LOOK):
            cpA(p, p).start()
            cpB(p, p).start()

        neg = jnp.full((16,), -jnp.inf, dtype=jnp.float32)

        def reduce_chunk(buf, n, accs):
            def body(t, accs):
                a0, a1, a2, a3 = accs
                for u in range(8):
                    r = t * 8 + u
                    a0 = jnp.maximum(a0, buf[r, pl.ds(0, 16)])
                    a1 = jnp.maximum(a1, buf[r, pl.ds(16, 16)])
                    a2 = jnp.maximum(a2, buf[r, pl.ds(32, 16)])
                    a3 = jnp.maximum(a3, buf[r, pl.ds(48, 16)])
                return a0, a1, a2, a3
            return lax.fori_loop(0, n // 8, body, accs)

        def group(g, carry):
            for p in range(_LOOK):
                row = _LOOK * g + p
                cpA(row, p).wait()
                accs = reduce_chunk(bufA[p], _CA, (neg, neg, neg, neg))
                cpB(row, p).wait()
                accs = reduce_chunk(bufB[p], _CB, accs)

                @pl.when(row + _LOOK < _ROWS_PER_W)
                def _start_next():
                    cpA(row + _LOOK, p).start()
                    cpB(row + _LOOK, p).start()

                a0, a1, a2, a3 = accs
                outs_v[pl.ds(row * _EMB, 16)] = a0
                outs_v[pl.ds(row * _EMB + 16, 16)] = a1
                outs_v[pl.ds(row * _EMB + 32, 16)] = a2
                outs_v[pl.ds(row * _EMB + 48, 16)] = a3
            return carry

        lax.fori_loop(0, _ROWS_PER_W // _LOOK, group, 0)

        pltpu.sync_copy(outs_v,
                        out_hbm.at[pl.ds(wid * _ROWS_PER_W * _EMB,
                                         _ROWS_PER_W * _EMB)])

    return pool_kernel


def _tc_head(pooled, W_out, b_out):
    def body(x_ref, w_ref, b_ref, o_ref):
        logits = jnp.dot(x_ref[...], w_ref[...],
                         preferred_element_type=jnp.float32) + b_ref[...]
        m = jnp.max(logits, axis=-1, keepdims=True)
        e = jnp.exp(logits - m)
        o_ref[...] = e / jnp.sum(e, axis=-1, keepdims=True)

    return pl.pallas_call(
        body,
        out_shape=jax.ShapeDtypeStruct((_BATCH, _NOUT), jnp.float32),
    )(pooled, W_out, b_out.reshape(1, _NOUT))


def kernel(indices, emb_table, W_out, b_out):
    idxA = indices[:, :_CA]
    idxB = jnp.pad(indices[:, _CA:], ((0, 0), (0, _CA - _CB)))
    pooled_flat = _sc_pool(idxA, emb_table)(idxA, idxB, emb_table)
    pooled = pooled_flat.reshape(_BATCH, _EMB)
    return _tc_head(pooled, W_out, b_out)
